# Initial kernel scaffold; baseline (speedup 1.0000x reference)
import jax
import jax.numpy as jnp
from jax.experimental import pallas as pl


def kernel(x, attn, imgSp, v_w, v_b, proj_w, proj_b):
    raise NotImplementedError("write your pallas kernel here")


# speedup vs baseline: 5.6496x; 5.6496x over previous
import functools

import jax
import jax.numpy as jnp
from jax import lax
from jax.experimental import pallas as pl
from jax.experimental.pallas import tpu as pltpu
from jax.experimental.pallas import tpu_sc as plsc

H = 56
W = 56
C = 192
NH = 4
HD = C // NH
KS = 7
P = H * W
NC = 2
NS = 16
NW = NC * NS


def _vproj_body(x_ref, w_ref, b_ref, sp_ref, u_ref):
    v = jnp.dot(x_ref[...], w_ref[...], preferred_element_type=jnp.float32)
    u_ref[...] = (v + b_ref[...]) * sp_ref[...]


def _oproj_body(a_ref, w_ref, b_ref, o_ref):
    o_ref[...] = (
        jnp.dot(a_ref[...], w_ref[...], preferred_element_type=jnp.float32)
        + b_ref[...]
    )


_agg_mesh = plsc.VectorSubcoreMesh(core_axis_name="c", subcore_axis_name="s")


@functools.partial(
    pl.kernel,
    out_type=jax.ShapeDtypeStruct((P, C), jnp.float32),
    mesh=_agg_mesh,
    scratch_types=[
        pltpu.VMEM((KS, W, C), jnp.float32),
        pltpu.VMEM((W, 208), jnp.float32),
        pltpu.VMEM((W, C), jnp.float32),
    ],
)
def _agg(u_hbm, attn_hbm, out_hbm, u_buf, a_buf, o_buf):
    wid = lax.axis_index("s") * NC + lax.axis_index("c")

    def do_row(h):
        rs = jnp.clip(h - (KS // 2), 0, H - KS)
        pltpu.sync_copy(u_hbm.at[pl.ds(rs, KS)], u_buf)
        pltpu.sync_copy(attn_hbm.at[h], a_buf)

        def wbody(w, carry):
            cs = jnp.clip(w - (KS // 2), 0, W - KS)
            for n in range(NH):
                lo = (n * KS * KS) // 16
                hi = (n * KS * KS + KS * KS - 1) // 16
                avecs = {
                    i: a_buf[w, pl.ds(i * 16, 16)] for i in range(lo, hi + 1)
                }
                acc = [jnp.zeros((16,), jnp.float32) for _ in range(3)]
                for k in range(KS):
                    for l in range(KS):
                        col = cs + l
                        idx = n * KS * KS + k * KS + l
                        av = jnp.broadcast_to(avecs[idx // 16][idx % 16], (16,))
                        for j in range(3):
                            acc[j] = acc[j] + av * u_buf[
                                k, col, pl.ds((n * 3 + j) * 16, 16)
                            ]
                for j in range(3):
                    o_buf[w, pl.ds((n * 3 + j) * 16, 16)] = acc[j]
            return carry

        lax.fori_loop(0, W, wbody, 0)
        pltpu.sync_copy(o_buf, out_hbm.at[pl.ds(h * W, W)])

    def rowbody(i, carry):
        h = wid + i * NW

        @pl.when(h < H)
        def _():
            do_row(h)

        return carry

    lax.fori_loop(0, 2, rowbody, 0)


def kernel(x, attn, imgSp, v_w, v_b, proj_w, proj_b):
    x2d = x.reshape(P, C)
    sp2d = imgSp.reshape(P, 1)
    attn_t = attn.reshape(NH, H, W, KS * KS).transpose(1, 2, 0, 3)
    attn_t = attn_t.reshape(H, W, NH * KS * KS)
    attn_t = jnp.pad(attn_t, ((0, 0), (0, 0), (0, 208 - NH * KS * KS)))

    u = pl.pallas_call(
        _vproj_body,
        out_shape=jax.ShapeDtypeStruct((P, C), jnp.float32),
    )(x2d, v_w.T, v_b.reshape(1, C), sp2d)

    agg = _agg(u.reshape(H, W, C), attn_t)

    out = pl.pallas_call(
        _oproj_body,
        out_shape=jax.ShapeDtypeStruct((P, C), jnp.float32),
    )(agg, proj_w.T, proj_b.reshape(1, C))
    return out.reshape(1, H, W, C)
